# Initial kernel scaffold; baseline (speedup 1.0000x reference)
import functools

import jax
import jax.numpy as jnp
from jax import lax
from jax.experimental import pallas as pl
from jax.experimental.pallas import tpu as pltpu
from jax.experimental.pallas import tpu_sc as plsc

N = 10000
E = 320000
F = 128
H = 8
D = 16
NC = 2
NS = 16
NW = NC * NS
EPW = E // NW
C = 80
NCHUNK = EPW // C
RPT = N // NS

R = 500
G = N // R


def _pre_body(x_ref, w_ref, alm_ref, arm_ref, feat_ref, el_ref, er_ref):
    f = jnp.dot(x_ref[...], w_ref[...], preferred_element_type=jnp.float32)
    feat_ref[...] = f
    el_ref[...] = jnp.dot(f, alm_ref[...], preferred_element_type=jnp.float32)
    er_ref[...] = jnp.dot(f, arm_ref[...], preferred_element_type=jnp.float32)


def _tc_pre(x, w, alm, arm):
    return pl.pallas_call(
        _pre_body,
        grid=(G,),
        in_specs=[pl.BlockSpec((R, F), lambda i: (i, 0)),
                  pl.BlockSpec((F, F), lambda i: (0, 0)),
                  pl.BlockSpec((F, 16), lambda i: (0, 0)),
                  pl.BlockSpec((F, 16), lambda i: (0, 0))],
        out_specs=[pl.BlockSpec((R, F), lambda i: (i, 0)),
                   pl.BlockSpec((R, 16), lambda i: (i, 0)),
                   pl.BlockSpec((R, 16), lambda i: (i, 0))],
        out_shape=[jax.ShapeDtypeStruct((N, F), jnp.float32),
                   jax.ShapeDtypeStruct((N, 16), jnp.float32),
                   jax.ShapeDtypeStruct((N, 16), jnp.float32)],
    )(x, w, alm, arm)


def _normalize(a0, a1, d0, d1, hp, b):
    num = a0 + a1
    den = d0 + d1
    expand = (lax.broadcasted_iota(jnp.int32, (16, F), 1) // D ==
              lax.broadcasted_iota(jnp.int32, (16, F), 0)).astype(jnp.float32)
    dfull = jnp.dot(den, expand, preferred_element_type=jnp.float32)
    return num / jnp.maximum(dfull, 1e-9) + hp + b


def _mid_body(a0, a1, d0, d1, hp, b, w, alm, arm,
              h1_ref, feat_ref, el_ref, er_ref):
    rst = _normalize(a0[...], a1[...], d0[...], d1[...], hp[...], b[...])
    rst = jnp.where(rst > 0, rst, 0.01 * rst)
    h1_ref[...] = rst
    f = jnp.dot(rst, w[...], preferred_element_type=jnp.float32)
    feat_ref[...] = f
    el_ref[...] = jnp.dot(f, alm[...], preferred_element_type=jnp.float32)
    er_ref[...] = jnp.dot(f, arm[...], preferred_element_type=jnp.float32)


def _tc_mid(a0, a1, d0, d1, hp, b, w, alm, arm):
    big = pl.BlockSpec((R, F), lambda i: (i, 0))
    sml = pl.BlockSpec((R, 16), lambda i: (i, 0))
    return pl.pallas_call(
        _mid_body,
        grid=(G,),
        in_specs=[big, big, sml, sml, big,
                  pl.BlockSpec((1, F), lambda i: (0, 0)),
                  pl.BlockSpec((F, F), lambda i: (0, 0)),
                  pl.BlockSpec((F, 16), lambda i: (0, 0)),
                  pl.BlockSpec((F, 16), lambda i: (0, 0))],
        out_specs=[big, big, sml, sml],
        out_shape=[jax.ShapeDtypeStruct((N, F), jnp.float32),
                   jax.ShapeDtypeStruct((N, F), jnp.float32),
                   jax.ShapeDtypeStruct((N, 16), jnp.float32),
                   jax.ShapeDtypeStruct((N, 16), jnp.float32)],
    )(a0, a1, d0, d1, hp, b, w, alm, arm)


def _fin_body(a0, a1, d0, d1, hp, b, out_ref):
    out_ref[...] = _normalize(a0[...], a1[...], d0[...], d1[...],
                              hp[...], b[...])


def _tc_fin(a0, a1, d0, d1, hp, b):
    big = pl.BlockSpec((R, F), lambda i: (i, 0))
    sml = pl.BlockSpec((R, 16), lambda i: (i, 0))
    return pl.pallas_call(
        _fin_body,
        grid=(G,),
        in_specs=[big, big, sml, sml, big,
                  pl.BlockSpec((1, F), lambda i: (0, 0))],
        out_specs=big,
        out_shape=jax.ShapeDtypeStruct((N, F), jnp.float32),
    )(a0, a1, d0, d1, hp, b)


def _edge_body(feat_h, el_h, er_h, src_h, dst_h, z128_h, z16_h,
               acc_o, den_o,
               src_v, dst_v, els_v, erd_v, feats_v, msg_v, den_v,
               acc_sh, den_sh, sem):
    c = lax.axis_index("c")
    s = lax.axis_index("s")
    r0 = s * RPT
    pltpu.sync_copy(z128_h.at[pl.ds(r0, RPT)], acc_sh.at[pl.ds(r0, RPT)])
    pltpu.sync_copy(z16_h.at[pl.ds(r0, RPT)], den_sh.at[pl.ds(r0, RPT)])
    plsc.subcore_barrier()

    base = (c * NS + s) * EPW

    def chunk(i, carry):
        off = base + i * C
        pltpu.sync_copy(src_h.at[pl.ds(off, C)], src_v)
        pltpu.sync_copy(dst_h.at[pl.ds(off, C)], dst_v)
        pltpu.async_copy(el_h.at[src_v], els_v, sem).wait()
        pltpu.async_copy(er_h.at[dst_v], erd_v, sem).wait()
        pltpu.async_copy(feat_h.at[src_v], feats_v, sem).wait()

        def edge(j, carry2):
            ev = els_v[j, :] + erd_v[j, :]
            ev = jnp.where(ev > 0.0, ev, 0.2 * ev)
            ex = jnp.exp(ev)
            den_v[j, :] = ex
            for h in range(H):
                sx = den_v[j, h]
                msg_v[j, pl.ds(h * D, D)] = feats_v[j, pl.ds(h * D, D)] * sx
            return carry2

        lax.fori_loop(0, C, edge, 0)
        pltpu.sync_copy(msg_v, acc_sh.at[dst_v], add=True)
        pltpu.sync_copy(den_v, den_sh.at[dst_v], add=True)
        return carry

    lax.fori_loop(0, NCHUNK, chunk, 0)
    plsc.subcore_barrier()
    pltpu.sync_copy(acc_sh.at[pl.ds(r0, RPT)], acc_o.at[c, pl.ds(r0, RPT)])
    pltpu.sync_copy(den_sh.at[pl.ds(r0, RPT)], den_o.at[c, pl.ds(r0, RPT)])


def _edge(feat, el16, er16, src, dst, z128, z16):
    mesh = plsc.VectorSubcoreMesh(core_axis_name="c", subcore_axis_name="s",
                                  num_cores=NC, num_subcores=NS)
    run = pl.kernel(
        _edge_body,
        out_type=(jax.ShapeDtypeStruct((NC, N, F), jnp.float32),
                  jax.ShapeDtypeStruct((NC, N, 16), jnp.float32)),
        mesh=mesh,
        scratch_types=[
            pltpu.VMEM((C,), jnp.int32),
            pltpu.VMEM((C,), jnp.int32),
            pltpu.VMEM((C, 16), jnp.float32),
            pltpu.VMEM((C, 16), jnp.float32),
            pltpu.VMEM((C, F), jnp.float32),
            pltpu.VMEM((C, F), jnp.float32),
            pltpu.VMEM((C, 16), jnp.float32),
            pltpu.VMEM_SHARED((N, F), jnp.float32),
            pltpu.VMEM_SHARED((N, 16), jnp.float32),
            pltpu.SemaphoreType.DMA,
        ],
    )
    return run(feat, el16, er16, src, dst, z128, z16)


def kernel(n_feat, edge_index, W0, al0, ar0, b0, W1, al1, ar1, b1):
    src = edge_index[0].astype(jnp.int32)
    dst = edge_index[1].astype(jnp.int32)
    eye = (jnp.arange(F)[:, None] // D ==
           jnp.arange(16)[None, :]).astype(jnp.float32)
    alm0 = al0.reshape(-1)[:, None] * eye
    arm0 = ar0.reshape(-1)[:, None] * eye
    alm1 = al1.reshape(-1)[:, None] * eye
    arm1 = ar1.reshape(-1)[:, None] * eye
    z128 = jnp.zeros((N, F), jnp.float32)
    z16 = jnp.zeros((N, 16), jnp.float32)

    feat1, el1, er1 = _tc_pre(n_feat, W0, alm0, arm0)
    acc1, den1 = _edge(feat1, el1, er1, src, dst, z128, z16)
    h1, feat2, el2, er2 = _tc_mid(acc1[0], acc1[1], den1[0], den1[1],
                                  n_feat, b0.reshape(1, F), W1, alm1, arm1)
    acc2, den2 = _edge(feat2, el2, er2, src, dst, z128, z16)
    out = _tc_fin(acc2[0], acc2[1], den2[0], den2[1], h1, b1.reshape(1, F))
    return out


# speedup vs baseline: 38.3611x; 38.3611x over previous
import functools

import jax
import jax.numpy as jnp
from jax import lax
from jax.experimental import pallas as pl
from jax.experimental.pallas import tpu as pltpu
from jax.experimental.pallas import tpu_sc as plsc

N = 10000
E = 320000
F = 128
H = 8
D = 16
NC = 2
NS = 16
NW = NC * NS
EPW = E // NW
C = 80
NCHUNK = EPW // C
NP = 10240
RPT = NP // NS

R = 1000
G = N // R


def _pre_body(x_ref, w_ref, alm_ref, arm_ref, feat_ref, el_ref, er_ref):
    f = jnp.dot(x_ref[...], w_ref[...], preferred_element_type=jnp.float32)
    feat_ref[...] = f
    el_ref[...] = jnp.dot(f, alm_ref[...], preferred_element_type=jnp.float32)
    er_ref[...] = jnp.dot(f, arm_ref[...], preferred_element_type=jnp.float32)


def _tc_pre(x, w, alm, arm):
    return pl.pallas_call(
        _pre_body,
        grid=(G,),
        in_specs=[pl.BlockSpec((R, F), lambda i: (i, 0)),
                  pl.BlockSpec((F, F), lambda i: (0, 0)),
                  pl.BlockSpec((F, 16), lambda i: (0, 0)),
                  pl.BlockSpec((F, 16), lambda i: (0, 0))],
        out_specs=[pl.BlockSpec((R, F), lambda i: (i, 0)),
                   pl.BlockSpec((R, 16), lambda i: (i, 0)),
                   pl.BlockSpec((R, 16), lambda i: (i, 0))],
        out_shape=[jax.ShapeDtypeStruct((N, F), jnp.float32),
                   jax.ShapeDtypeStruct((N, 16), jnp.float32),
                   jax.ShapeDtypeStruct((N, 16), jnp.float32)],
    )(x, w, alm, arm)


def _normalize(a0, a1, d0, d1, hp, b):
    num = a0 + a1
    den = d0 + d1
    expand = (lax.broadcasted_iota(jnp.int32, (16, F), 1) // D ==
              lax.broadcasted_iota(jnp.int32, (16, F), 0)).astype(jnp.float32)
    dfull = jnp.dot(den, expand, preferred_element_type=jnp.float32)
    return num / jnp.maximum(dfull, 1e-9) + hp + b


def _mid_body(a0, a1, d0, d1, hp, b, w, alm, arm,
              h1_ref, feat_ref, el_ref, er_ref):
    rst = _normalize(a0[...], a1[...], d0[...], d1[...], hp[...], b[...])
    rst = jnp.where(rst > 0, rst, 0.01 * rst)
    h1_ref[...] = rst
    f = jnp.dot(rst, w[...], preferred_element_type=jnp.float32)
    feat_ref[...] = f
    el_ref[...] = jnp.dot(f, alm[...], preferred_element_type=jnp.float32)
    er_ref[...] = jnp.dot(f, arm[...], preferred_element_type=jnp.float32)


def _tc_mid(a0, a1, d0, d1, hp, b, w, alm, arm):
    big = pl.BlockSpec((R, F), lambda i: (i, 0))
    sml = pl.BlockSpec((R, 16), lambda i: (i, 0))
    return pl.pallas_call(
        _mid_body,
        grid=(G,),
        in_specs=[big, big, sml, sml, big,
                  pl.BlockSpec((1, F), lambda i: (0, 0)),
                  pl.BlockSpec((F, F), lambda i: (0, 0)),
                  pl.BlockSpec((F, 16), lambda i: (0, 0)),
                  pl.BlockSpec((F, 16), lambda i: (0, 0))],
        out_specs=[big, big, sml, sml],
        out_shape=[jax.ShapeDtypeStruct((N, F), jnp.float32),
                   jax.ShapeDtypeStruct((N, F), jnp.float32),
                   jax.ShapeDtypeStruct((N, 16), jnp.float32),
                   jax.ShapeDtypeStruct((N, 16), jnp.float32)],
    )(a0, a1, d0, d1, hp, b, w, alm, arm)


def _fin_body(a0, a1, d0, d1, hp, b, out_ref):
    out_ref[...] = _normalize(a0[...], a1[...], d0[...], d1[...],
                              hp[...], b[...])


def _tc_fin(a0, a1, d0, d1, hp, b):
    big = pl.BlockSpec((R, F), lambda i: (i, 0))
    sml = pl.BlockSpec((R, 16), lambda i: (i, 0))
    return pl.pallas_call(
        _fin_body,
        grid=(G,),
        in_specs=[big, big, sml, sml, big,
                  pl.BlockSpec((1, F), lambda i: (0, 0))],
        out_specs=big,
        out_shape=jax.ShapeDtypeStruct((N, F), jnp.float32),
    )(a0, a1, d0, d1, hp, b)


def _edge_body(feat_h, el_h, er_h, src_h, dst_h, z128_h, z16_h,
               acc_o, den_o,
               src_v, dst_v, els_v, erd_v, feats_v, msg_v, den_v,
               acc_sh, den_sh, sem):
    c = lax.axis_index("c")
    s = lax.axis_index("s")
    r0 = s * RPT
    pltpu.sync_copy(z128_h.at[pl.ds(r0, RPT)], acc_sh.at[pl.ds(r0, RPT)])
    pltpu.sync_copy(z16_h.at[pl.ds(r0, RPT)], den_sh.at[pl.ds(r0, RPT)])
    plsc.subcore_barrier()

    base = (c * NS + s) * EPW

    def chunk(i, carry):
        off = base + i * C
        pltpu.sync_copy(src_h.at[pl.ds(off, C)], src_v)
        pltpu.sync_copy(dst_h.at[pl.ds(off, C)], dst_v)
        pltpu.async_copy(el_h.at[src_v], els_v, sem).wait()
        pltpu.async_copy(er_h.at[dst_v], erd_v, sem).wait()
        pltpu.async_copy(feat_h.at[src_v], feats_v, sem).wait()

        def edge(j, carry2):
            ev = els_v[j, :] + erd_v[j, :]
            ev = jnp.where(ev > 0.0, ev, 0.2 * ev)
            ex = jnp.exp(ev)
            den_v[j, :] = ex
            for h in range(H):
                sx = ex[h]
                msg_v[j, pl.ds(h * D, D)] = feats_v[j, pl.ds(h * D, D)] * sx
            return carry2

        lax.fori_loop(0, C, edge, 0)
        pltpu.sync_copy(msg_v, acc_sh.at[dst_v], add=True)
        pltpu.sync_copy(den_v, den_sh.at[dst_v], add=True)
        return carry

    lax.fori_loop(0, NCHUNK, chunk, 0)
    plsc.subcore_barrier()
    pltpu.sync_copy(acc_sh.at[pl.ds(r0, RPT)], acc_o.at[c, pl.ds(r0, RPT)])
    pltpu.sync_copy(den_sh.at[pl.ds(r0, RPT)], den_o.at[c, pl.ds(r0, RPT)])


def _edge(feat, el16, er16, src, dst, z128, z16):
    mesh = plsc.VectorSubcoreMesh(core_axis_name="c", subcore_axis_name="s",
                                  num_cores=NC, num_subcores=NS)
    run = pl.kernel(
        _edge_body,
        out_type=(jax.ShapeDtypeStruct((NC, NP, F), jnp.float32),
                  jax.ShapeDtypeStruct((NC, NP, 16), jnp.float32)),
        mesh=mesh,
        compiler_params=pltpu.CompilerParams(use_tc_tiling_on_sc=False),
        scratch_types=[
            pltpu.VMEM((C,), jnp.int32),
            pltpu.VMEM((C,), jnp.int32),
            pltpu.VMEM((C, 16), jnp.float32),
            pltpu.VMEM((C, 16), jnp.float32),
            pltpu.VMEM((C, F), jnp.float32),
            pltpu.VMEM((C, F), jnp.float32),
            pltpu.VMEM((C, 16), jnp.float32),
            pltpu.VMEM_SHARED((NP, F), jnp.float32),
            pltpu.VMEM_SHARED((NP, 16), jnp.float32),
            pltpu.SemaphoreType.DMA,
        ],
    )
    return run(feat, el16, er16, src, dst, z128, z16)


def kernel(n_feat, edge_index, W0, al0, ar0, b0, W1, al1, ar1, b1):
    src = edge_index[0].astype(jnp.int32)
    dst = edge_index[1].astype(jnp.int32)
    eye = (jnp.arange(F)[:, None] // D ==
           jnp.arange(16)[None, :]).astype(jnp.float32)
    alm0 = al0.reshape(-1)[:, None] * eye
    arm0 = ar0.reshape(-1)[:, None] * eye
    alm1 = al1.reshape(-1)[:, None] * eye
    arm1 = ar1.reshape(-1)[:, None] * eye
    z128 = jnp.zeros((NP, F), jnp.float32)
    z16 = jnp.zeros((NP, 16), jnp.float32)

    feat1, el1, er1 = _tc_pre(n_feat, W0, alm0, arm0)
    acc1, den1 = _edge(feat1, el1, er1, src, dst, z128, z16)
    acc1, den1 = acc1[:, :N], den1[:, :N]
    h1, feat2, el2, er2 = _tc_mid(acc1[0], acc1[1], den1[0], den1[1],
                                  n_feat, b0.reshape(1, F), W1, alm1, arm1)
    acc2, den2 = _edge(feat2, el2, er2, src, dst, z128, z16)
    acc2, den2 = acc2[:, :N], den2[:, :N]
    out = _tc_fin(acc2[0], acc2[1], den2[0], den2[1], h1, b1.reshape(1, F))
    return out
